# baseline (speedup 1.0000x reference)
import functools

import jax
import jax.numpy as jnp
from jax import lax
from jax.experimental import pallas as pl
from jax.experimental.pallas import tpu as pltpu
from jax.experimental.pallas import tpu_sc as plsc

N = 10000
E = 160000
D = 256
H = 128
NC = 2
NS = 16
CHUNK = 128
NCH = 80
DCH = NCH // NC
EPT = NCH * CHUNK
E_PAD = NS * EPT
N_PAD = 10112
RPT = N_PAD // NS


def _sc_aggregate(x2, src_t, dst_t, zrows, ones_blk):
    mesh = plsc.VectorSubcoreMesh(core_axis_name="c", subcore_axis_name="s",
                                  num_cores=NC, num_subcores=NS)

    @functools.partial(
        pl.kernel,
        out_type=(
            jax.ShapeDtypeStruct((NC, N_PAD, H), jnp.float32),
            jax.ShapeDtypeStruct((NC, N_PAD, H), jnp.float32),
        ),
        mesh=mesh,
        scratch_types=[
            pltpu.VMEM((CHUNK,), jnp.int32),
            pltpu.VMEM((CHUNK,), jnp.int32),
            pltpu.VMEM((NCH, CHUNK), jnp.int32),
            pltpu.VMEM((CHUNK, H), jnp.float32),
            pltpu.VMEM((CHUNK, H), jnp.float32),
            pltpu.VMEM_SHARED((N_PAD, H), jnp.float32),
            pltpu.SemaphoreType.DMA,
            pltpu.SemaphoreType.DMA,
            pltpu.SemaphoreType.DMA,
            pltpu.SemaphoreType.DMA,
        ],
    )
    def k(x2_hbm, src_hbm, dst_hbm, z_hbm, ones_hbm, sums_hbm, degp_hbm,
          idx0, idx1, dst_v, rows0, rows1, acc_sh,
          semI0, semI1, semA, semB):
        cid = lax.axis_index("c")
        sid = lax.axis_index("s")
        nbase = sid * RPT

        pltpu.sync_copy(z_hbm, acc_sh.at[pl.ds(nbase, RPT)])

        pltpu.sync_copy(dst_hbm.at[sid], dst_v)

        def istart(c, ib, sem):
            o = pl.multiple_of(c * CHUNK, CHUNK)
            pltpu.async_copy(src_hbm.at[cid, sid, pl.ds(o, CHUNK)], ib, sem)

        def iwait(ib, sem):
            pltpu.make_async_copy(
                src_hbm.at[cid, sid, pl.ds(0, CHUNK)], ib, sem).wait()

        def gstart(ib, buf, sem):
            pltpu.async_copy(x2_hbm.at[ib], buf, sem)

        def gwait(buf, sem):
            pltpu.make_async_copy(z_hbm.at[pl.ds(0, CHUNK)], buf, sem).wait()

        plsc.subcore_barrier()

        istart(0, idx0, semI0)
        istart(1, idx1, semI1)
        iwait(idx0, semI0)
        gstart(idx0, rows0, semA)
        iwait(idx1, semI1)
        gstart(idx1, rows1, semB)

        def pair(i, carry):
            c0 = i * 2
            gwait(rows0, semA)

            @pl.when(c0 + 2 < NCH)
            def _():
                istart(c0 + 2, idx0, semI0)

            pltpu.sync_copy(rows0, acc_sh.at[dst_v.at[c0]], add=True)

            @pl.when(c0 + 2 < NCH)
            def _():
                iwait(idx0, semI0)
                gstart(idx0, rows0, semA)

            gwait(rows1, semB)

            @pl.when(c0 + 3 < NCH)
            def _():
                istart(c0 + 3, idx1, semI1)

            pltpu.sync_copy(rows1, acc_sh.at[dst_v.at[c0 + 1]], add=True)

            @pl.when(c0 + 3 < NCH)
            def _():
                iwait(idx1, semI1)
                gstart(idx1, rows1, semB)

            return carry

        lax.fori_loop(0, NCH // 2, pair, 0)

        plsc.subcore_barrier()
        pltpu.sync_copy(acc_sh.at[pl.ds(nbase, RPT)],
                        sums_hbm.at[cid, pl.ds(nbase, RPT)])
        pltpu.sync_copy(z_hbm, acc_sh.at[pl.ds(nbase, RPT)])
        pltpu.sync_copy(ones_hbm, rows0)
        plsc.subcore_barrier()

        def dchunk(c, carry):
            pltpu.async_copy(rows0, acc_sh.at[dst_v.at[c]], semA, add=True)
            return carry

        lax.fori_loop(cid * DCH, (cid + 1) * DCH, dchunk, 0)

        def ddrain(i, carry):
            gwait(rows0, semA)
            return carry

        lax.fori_loop(0, DCH, ddrain, 0)

        plsc.subcore_barrier()
        pltpu.sync_copy(acc_sh.at[pl.ds(nbase, RPT)],
                        degp_hbm.at[cid, pl.ds(nbase, RPT)])

    return k(x2, src_t, dst_t, zrows, ones_blk)


def _tc1_body(x_ref, w1_ref, b_ref, o_ref):
    o_ref[...] = jnp.dot(
        x_ref[...], w1_ref[...],
        preferred_element_type=jnp.float32) + b_ref[...]


def _tc1_xw1(x, w1t, b2):
    R = 1000
    return pl.pallas_call(
        _tc1_body,
        grid=(N // R,),
        in_specs=[
            pl.BlockSpec((R, D), lambda i: (i, 0)),
            pl.BlockSpec((D, D), lambda i: (0, 0)),
            pl.BlockSpec((1, D), lambda i: (0, 0)),
        ],
        out_specs=pl.BlockSpec((R, D), lambda i: (i, 0)),
        out_shape=jax.ShapeDtypeStruct((N, D), jnp.float32),
    )(x, w1t, b2)


def _tc2_body(y_ref, s_ref, d_ref, w2a_ref, w2b_ref, o_ref):
    a0 = s_ref[0]
    a1 = s_ref[1]
    deg = d_ref[0, :, 0:1] + d_ref[1, :, 0:1]
    r = 1.0 / jnp.maximum(deg, 1.0)
    acc = y_ref[...]
    acc = acc + jnp.dot(a0 * r, w2a_ref[...],
                        preferred_element_type=jnp.float32)
    acc = acc + jnp.dot(a1 * r, w2b_ref[...],
                        preferred_element_type=jnp.float32)
    o_ref[...] = acc


def _tc2_mean(y, sums, degp, w2a, w2b):
    R = 1000
    return pl.pallas_call(
        _tc2_body,
        grid=(N // R,),
        in_specs=[
            pl.BlockSpec((R, D), lambda i: (i, 0)),
            pl.BlockSpec((NC, R, H), lambda i: (0, i, 0)),
            pl.BlockSpec((NC, R, H), lambda i: (0, i, 0)),
            pl.BlockSpec((H, D), lambda i: (0, 0)),
            pl.BlockSpec((H, D), lambda i: (0, 0)),
        ],
        out_specs=pl.BlockSpec((R, D), lambda i: (i, 0)),
        out_shape=jax.ShapeDtypeStruct((N, D), jnp.float32),
    )(y, sums, degp, w2a, w2b)


def kernel(x, edge_index, W, b):
    src = edge_index[0]
    dst = edge_index[1]

    x2 = x.reshape(NC * N, H)

    src_p = jnp.concatenate([src, jnp.zeros((E_PAD - E,), jnp.int32)])
    src2 = src_p * 2
    src_t = jnp.stack([src2, src2 + 1]).reshape(NC, NS, EPT)
    dst_t = jnp.concatenate(
        [dst, jnp.full((E_PAD - E,), N, jnp.int32)]).reshape(NS, NCH, CHUNK)
    zrows = jnp.zeros((RPT, H), jnp.float32)
    ones_blk = jnp.ones((CHUNK, H), jnp.float32)

    w1t = W[:, :D].T
    w2a = W[:, D:D + H].T
    w2b = W[:, D + H:].T
    b2 = b.reshape(1, D)

    y = _tc1_xw1(x, w1t, b2)
    sums, degp = _sc_aggregate(x2, src_t, dst_t, zrows, ones_blk)
    return _tc2_mean(y, sums, degp, w2a, w2b)


# speedup vs baseline: 1.0012x; 1.0012x over previous
import functools

import jax
import jax.numpy as jnp
from jax import lax
from jax.experimental import pallas as pl
from jax.experimental.pallas import tpu as pltpu
from jax.experimental.pallas import tpu_sc as plsc

N = 10000
E = 160000
D = 256
H = 128
NC = 2
NS = 16
CHUNK = 128
NCH = 80
DCH = NCH // NC
EPT = NCH * CHUNK
E_PAD = NS * EPT
N_PAD = 10112
RPT = N_PAD // NS


def _sc_aggregate(x2, src_t, dst_t, zrows, ones_blk):
    mesh = plsc.VectorSubcoreMesh(core_axis_name="c", subcore_axis_name="s",
                                  num_cores=NC, num_subcores=NS)

    @functools.partial(
        pl.kernel,
        out_type=(
            jax.ShapeDtypeStruct((NC, N_PAD, H), jnp.float32),
            jax.ShapeDtypeStruct((NC, N_PAD, H), jnp.float32),
        ),
        mesh=mesh,
        scratch_types=[
            pltpu.VMEM((CHUNK,), jnp.int32),
            pltpu.VMEM((CHUNK,), jnp.int32),
            pltpu.VMEM((NCH, CHUNK), jnp.int32),
            pltpu.VMEM((CHUNK, H), jnp.float32),
            pltpu.VMEM((CHUNK, H), jnp.float32),
            pltpu.VMEM_SHARED((N_PAD, H), jnp.float32),
            pltpu.SemaphoreType.DMA,
            pltpu.SemaphoreType.DMA,
            pltpu.SemaphoreType.DMA,
            pltpu.SemaphoreType.DMA,
        ],
    )
    def k(x2_hbm, src_hbm, dst_hbm, z_hbm, ones_hbm, sums_hbm, degp_hbm,
          idx0, idx1, dst_v, rows0, rows1, acc_sh,
          semI0, semI1, semA, semB):
        cid = lax.axis_index("c")
        sid = lax.axis_index("s")
        nbase = sid * RPT

        pltpu.sync_copy(z_hbm, acc_sh.at[pl.ds(nbase, RPT)])

        pltpu.sync_copy(dst_hbm.at[sid], dst_v)

        def istart(c, ib, sem):
            o = pl.multiple_of(c * CHUNK, CHUNK)
            pltpu.async_copy(src_hbm.at[cid, sid, pl.ds(o, CHUNK)], ib, sem)

        def iwait(ib, sem):
            pltpu.make_async_copy(
                src_hbm.at[cid, sid, pl.ds(0, CHUNK)], ib, sem).wait()

        def gstart(ib, buf, sem):
            pltpu.async_copy(x2_hbm.at[ib], buf, sem)

        def gwait(buf, sem):
            pltpu.make_async_copy(z_hbm.at[pl.ds(0, CHUNK)], buf, sem).wait()

        plsc.subcore_barrier()

        istart(0, idx0, semI0)
        istart(1, idx1, semI1)
        iwait(idx0, semI0)
        gstart(idx0, rows0, semA)
        iwait(idx1, semI1)
        gstart(idx1, rows1, semB)

        def pair(i, carry):
            c0 = i * 2
            gwait(rows0, semA)

            @pl.when(c0 + 2 < NCH)
            def _():
                istart(c0 + 2, idx0, semI0)

            pltpu.sync_copy(rows0, acc_sh.at[dst_v.at[c0]], add=True)

            @pl.when(c0 + 2 < NCH)
            def _():
                iwait(idx0, semI0)
                gstart(idx0, rows0, semA)

            gwait(rows1, semB)

            @pl.when(c0 + 3 < NCH)
            def _():
                istart(c0 + 3, idx1, semI1)

            pltpu.sync_copy(rows1, acc_sh.at[dst_v.at[c0 + 1]], add=True)

            @pl.when(c0 + 3 < NCH)
            def _():
                iwait(idx1, semI1)
                gstart(idx1, rows1, semB)

            return carry

        lax.fori_loop(0, NCH // 2, pair, 0)

        plsc.subcore_barrier()
        pltpu.sync_copy(acc_sh.at[pl.ds(nbase, RPT)],
                        sums_hbm.at[cid, pl.ds(nbase, RPT)])
        pltpu.sync_copy(z_hbm, acc_sh.at[pl.ds(nbase, RPT)])
        pltpu.sync_copy(ones_hbm, rows0)
        plsc.subcore_barrier()

        def dchunk(c, carry):
            pltpu.async_copy(rows0, acc_sh.at[dst_v.at[c]], semA, add=True)
            return carry

        lax.fori_loop(cid * DCH, (cid + 1) * DCH, dchunk, 0)

        def ddrain(i, carry):
            gwait(rows0, semA)
            return carry

        lax.fori_loop(0, DCH, ddrain, 0)

        plsc.subcore_barrier()
        pltpu.sync_copy(acc_sh.at[pl.ds(nbase, RPT)],
                        degp_hbm.at[cid, pl.ds(nbase, RPT)])

    return k(x2, src_t, dst_t, zrows, ones_blk)


def _tc_body(x_ref, s_ref, d_ref, w1_ref, w2a_ref, w2b_ref, b_ref, o_ref):
    a0 = s_ref[0]
    a1 = s_ref[1]
    deg = d_ref[0, :, 0:1] + d_ref[1, :, 0:1]
    r = 1.0 / jnp.maximum(deg, 1.0)
    acc = jnp.dot(x_ref[...], w1_ref[...], preferred_element_type=jnp.float32)
    acc = acc + jnp.dot(a0 * r, w2a_ref[...],
                        preferred_element_type=jnp.float32)
    acc = acc + jnp.dot(a1 * r, w2b_ref[...],
                        preferred_element_type=jnp.float32)
    o_ref[...] = acc + b_ref[...]


def _tc_linear(x, sums, degp, w1t, w2a, w2b, b2):
    R = 1000
    return pl.pallas_call(
        _tc_body,
        grid=(N // R,),
        in_specs=[
            pl.BlockSpec((R, D), lambda i: (i, 0)),
            pl.BlockSpec((NC, R, H), lambda i: (0, i, 0)),
            pl.BlockSpec((NC, R, H), lambda i: (0, i, 0)),
            pl.BlockSpec((D, D), lambda i: (0, 0)),
            pl.BlockSpec((H, D), lambda i: (0, 0)),
            pl.BlockSpec((H, D), lambda i: (0, 0)),
            pl.BlockSpec((1, D), lambda i: (0, 0)),
        ],
        out_specs=pl.BlockSpec((R, D), lambda i: (i, 0)),
        out_shape=jax.ShapeDtypeStruct((N, D), jnp.float32),
    )(x, sums, degp, w1t, w2a, w2b, b2)


def kernel(x, edge_index, W, b):
    src = edge_index[0]
    dst = edge_index[1]

    x2 = x.reshape(NC * N, H)

    src_p = jnp.concatenate([src, jnp.zeros((E_PAD - E,), jnp.int32)])
    src2 = src_p * 2
    src_t = jnp.stack([src2, src2 + 1]).reshape(NC, NS, EPT)
    dst_t = jnp.concatenate(
        [dst, jnp.full((E_PAD - E,), N, jnp.int32)]).reshape(NS, NCH, CHUNK)
    zrows = jnp.zeros((RPT, H), jnp.float32)
    ones_blk = jnp.ones((CHUNK, H), jnp.float32)

    sums, degp = _sc_aggregate(x2, src_t, dst_t, zrows, ones_blk)

    w1t = W[:, :D].T
    w2a = W[:, D:D + H].T
    w2b = W[:, D + H:].T
    b2 = b.reshape(1, D)
    return _tc_linear(x, sums, degp, w1t, w2a, w2b, b2)
